# baseline (speedup 1.0000x reference)
import functools

import jax
import jax.numpy as jnp
from jax import lax
from jax.experimental import pallas as pl
from jax.experimental.pallas import tpu as pltpu
from jax.experimental.pallas import tpu_sc as plsc

_NF = 26
_FD = 16
_NPAIR = (_NF * (_NF - 1)) // 2
_ROW_IN = _NF * _FD
_ROW_OUT = _NPAIR * 3 * _FD
_BATCH = 4096
_NW = 32
_PER_W = _BATCH // _NW


def _pin_body(x_hbm, out_hbm, xbuf, obuf):
    wid = lax.axis_index("s") * 2 + lax.axis_index("c")
    base = wid * _PER_W

    def step(s, carry):
        b = base + s
        pltpu.sync_copy(x_hbm.at[b], xbuf)
        a = [xbuf[pl.ds(f * _FD, _FD)] for f in range(_NF)]
        p = 0
        for i in range(_NF - 1):
            ai = a[i]
            for j in range(i + 1, _NF):
                off = p * 3 * _FD
                obuf[pl.ds(off, _FD)] = ai
                obuf[pl.ds(off + _FD, _FD)] = a[j]
                obuf[pl.ds(off + 2 * _FD, _FD)] = ai * a[j]
                p += 1
        pltpu.sync_copy(obuf, out_hbm.at[b])
        return carry

    lax.fori_loop(0, _PER_W, step, 0)


@jax.jit
def kernel(x):
    xf = x.reshape(_BATCH, _ROW_IN)
    run = pl.kernel(
        _pin_body,
        out_type=jax.ShapeDtypeStruct((_BATCH, _ROW_OUT), jnp.float32),
        scratch_types=[
            pltpu.VMEM((_ROW_IN,), jnp.float32),
            pltpu.VMEM((_ROW_OUT,), jnp.float32),
        ],
        mesh=plsc.VectorSubcoreMesh(core_axis_name="c", subcore_axis_name="s"),
    )
    out = run(xf)
    return out.reshape(_BATCH, _NPAIR, 3 * _FD)


# speedup vs baseline: 1.1266x; 1.1266x over previous
import jax
import jax.numpy as jnp
from jax import lax
from jax.experimental import pallas as pl
from jax.experimental.pallas import tpu as pltpu
from jax.experimental.pallas import tpu_sc as plsc

_NF = 26
_FD = 16
_NPAIR = (_NF * (_NF - 1)) // 2
_ROW_IN = _NF * _FD
_ROW_OUT = _NPAIR * 3 * _FD
_BATCH = 4096
_NW = 32
_PER_W = _BATCH // _NW


def _pin_body(x_hbm, out_hbm, xblk, obuf0, obuf1, sem0, sem1):
    wid = lax.axis_index("s") * 2 + lax.axis_index("c")
    base = wid * _PER_W

    pltpu.sync_copy(x_hbm.at[pl.ds(base, _PER_W)], xblk)

    obufs = (obuf0, obuf1)
    sems = (sem0, sem1)

    def compute_row(s, obuf):
        a = [xblk[s, pl.ds(f * _FD, _FD)] for f in range(_NF)]
        p = 0
        for i in range(_NF - 1):
            ai = a[i]
            for j in range(i + 1, _NF):
                off = p * 3 * _FD
                obuf[pl.ds(off, _FD)] = ai
                obuf[pl.ds(off + _FD, _FD)] = a[j]
                obuf[pl.ds(off + 2 * _FD, _FD)] = ai * a[j]
                p += 1

    def step(t, carry):
        for k in range(2):
            s = 2 * t + k

            @pl.when(t > 0)
            def _wait_prev():
                pltpu.make_async_copy(
                    obufs[k], out_hbm.at[base + s - 2], sems[k]).wait()

            compute_row(s, obufs[k])
            pltpu.async_copy(obufs[k], out_hbm.at[base + s], sems[k])
        return carry

    lax.fori_loop(0, _PER_W // 2, step, 0)

    for k in range(2):
        pltpu.make_async_copy(
            obufs[k], out_hbm.at[base + _PER_W - 2 + k], sems[k]).wait()


@jax.jit
def kernel(x):
    xf = x.reshape(_BATCH, _ROW_IN)
    run = pl.kernel(
        _pin_body,
        out_type=jax.ShapeDtypeStruct((_BATCH, _ROW_OUT), jnp.float32),
        scratch_types=[
            pltpu.VMEM((_PER_W, _ROW_IN), jnp.float32),
            pltpu.VMEM((_ROW_OUT,), jnp.float32),
            pltpu.VMEM((_ROW_OUT,), jnp.float32),
            pltpu.SemaphoreType.DMA,
            pltpu.SemaphoreType.DMA,
        ],
        mesh=plsc.VectorSubcoreMesh(core_axis_name="c", subcore_axis_name="s"),
    )
    out = run(xf)
    return out.reshape(_BATCH, _NPAIR, 3 * _FD)


# speedup vs baseline: 1.9979x; 1.7734x over previous
import jax
import jax.numpy as jnp
from jax import lax
from jax.experimental import pallas as pl
from jax.experimental.pallas import tpu as pltpu
from jax.experimental.pallas import tpu_sc as plsc

_NF = 26
_FD = 16
_NPAIR = (_NF * (_NF - 1)) // 2
_ROW_IN = _NF * _FD
_ROW_OUT = _NPAIR * 3 * _FD
_BATCH = 4096
_NW = 32
_LANES = _BATCH // _NW
_NSUB = _LANES // 16


def _pin_body(xt_hbm, out_hbm, xblk, obuf0, obuf1, sem0, sem1):
    wid = lax.axis_index("s") * 2 + lax.axis_index("c")
    lane0 = wid * _LANES

    pltpu.sync_copy(xt_hbm.at[:, pl.ds(lane0, _LANES)], xblk)

    obufs = (obuf0, obuf1)
    sems = (sem0, sem1)

    def compute_pair(obuf, ir, jr):
        for c in range(_FD):
            for u in range(_NSUB):
                sl = pl.ds(16 * u, 16)
                av = xblk[ir + c, sl]
                bv = xblk[jr + c, sl]
                obuf[c, sl] = av
                obuf[_FD + c, sl] = bv
                obuf[2 * _FD + c, sl] = av * bv

    def seg(i, carry):
        seg_base = (i * (2 * _NF - 1 - i)) // 2

        def pairj(j, carry2):
            p = seg_base + (j - i - 1)
            slot = lax.rem(p, 2)
            ir = _FD * i
            jr = _FD * j
            for k in range(2):
                @pl.when(slot == k)
                def _run(k=k):
                    @pl.when(p >= 2)
                    def _drain():
                        pltpu.make_async_copy(
                            obufs[k],
                            out_hbm.at[pl.ds(0, 3 * _FD), pl.ds(lane0, _LANES)],
                            sems[k]).wait()

                    compute_pair(obufs[k], ir, jr)
                    pltpu.async_copy(
                        obufs[k],
                        out_hbm.at[pl.ds(3 * _FD * p, 3 * _FD),
                                   pl.ds(lane0, _LANES)],
                        sems[k])
            return carry2

        return lax.fori_loop(i + 1, _NF, pairj, carry)

    lax.fori_loop(0, _NF - 1, seg, 0)

    for k in range(2):
        pltpu.make_async_copy(
            obufs[k],
            out_hbm.at[pl.ds(0, 3 * _FD), pl.ds(lane0, _LANES)],
            sems[k]).wait()


@jax.jit
def kernel(x):
    xt = x.transpose(1, 2, 0).reshape(_ROW_IN, _BATCH)
    run = pl.kernel(
        _pin_body,
        out_type=jax.ShapeDtypeStruct((_ROW_OUT, _BATCH), jnp.float32),
        scratch_types=[
            pltpu.VMEM((_ROW_IN, _LANES), jnp.float32),
            pltpu.VMEM((3 * _FD, _LANES), jnp.float32),
            pltpu.VMEM((3 * _FD, _LANES), jnp.float32),
            pltpu.SemaphoreType.DMA,
            pltpu.SemaphoreType.DMA,
        ],
        mesh=plsc.VectorSubcoreMesh(core_axis_name="c", subcore_axis_name="s"),
    )
    out_t = run(xt)
    return out_t.reshape(_NPAIR, 3 * _FD, _BATCH).transpose(2, 0, 1)
